# Initial kernel scaffold; baseline (speedup 1.0000x reference)
import jax
import jax.numpy as jnp
from jax.experimental import pallas as pl


def kernel(x_building, x_cable_group, x_transformer, edge_index_b2c, edge_index_c2t, edge_index_b2b, W_src_bl, W_dst_bl, att_src_bl, att_dst_bl, bias_bl, W_src_lt, W_dst_lt, att_src_lt, att_dst_lt, bias_lt, W_gcn, b_gcn):
    raise NotImplementedError("write your pallas kernel here")


# speedup vs baseline: 11.1383x; 11.1383x over previous
import functools

import jax
import jax.numpy as jnp
from jax import lax
from jax.experimental import pallas as pl
from jax.experimental.pallas import tpu as pltpu
from jax.experimental.pallas import tpu_sc as plsc

HID = 128
HEADS = 4
CH = 32
NB = 100000
NC = 10000
NT = 1000

NCORES = 2
NSUB = 16
NTILES = NCORES * NSUB

E1 = 100000
EPT1 = 3200
NBLK1 = EPT1 // 128
EPAD1 = EPT1 * NTILES
ND8_C = 10112
SHARE_C = ND8_C // NSUB

E3 = 10000
EPT3 = 384
NBLK3 = EPT3 // 128
EPAD3 = EPT3 * NTILES
ND8_T = 1024
SHARE_T = ND8_T // NSUB

E2 = 600000
EPT2 = 18816
NV2 = EPT2 // 16
EPAD2 = EPT2 * NTILES
NPASS = 14
CHUNK = 7152
CHR = 7168
GTRASH = 7152
SHARE_G = CHR // NSUB
CPK_CAP = EPT2 + 128

DEG_N = 100096
SHARE_D = DEG_N // NSUB


def _prep_b_body(xb_ref, dis_ref, ws_ref, as_ref, wg_ref, hs_ref, a16_ref,
                 xln_ref):
  xb = xb_ref[...]
  hs = jnp.dot(xb, ws_ref[...], preferred_element_type=jnp.float32)
  hs_ref[...] = hs
  a16_ref[...] = jnp.dot(hs, as_ref[...], preferred_element_type=jnp.float32)
  xl = jnp.dot(xb, wg_ref[...], preferred_element_type=jnp.float32)
  xln_ref[...] = xl * dis_ref[...]


def _tc_prep_b(xb, dis, wsrc, a_s, wgcn):
  br = 1000
  return pl.pallas_call(
      _prep_b_body,
      grid=(NB // br,),
      in_specs=[
          pl.BlockSpec((br, HID), lambda i: (i, 0)),
          pl.BlockSpec((br, 1), lambda i: (i, 0)),
          pl.BlockSpec((HID, HID), lambda i: (0, 0)),
          pl.BlockSpec((HID, 16), lambda i: (0, 0)),
          pl.BlockSpec((HID, HID), lambda i: (0, 0)),
      ],
      out_specs=[
          pl.BlockSpec((br, HID), lambda i: (i, 0)),
          pl.BlockSpec((br, 16), lambda i: (i, 0)),
          pl.BlockSpec((br, HID), lambda i: (i, 0)),
      ],
      out_shape=[
          jax.ShapeDtypeStruct((NB, HID), jnp.float32),
          jax.ShapeDtypeStruct((NB, 16), jnp.float32),
          jax.ShapeDtypeStruct((NB, HID), jnp.float32),
      ],
  )(xb, dis, wsrc, a_s, wgcn)


def _attdst_body(x_ref, w_ref, a_ref, o_ref):
  h = jnp.dot(x_ref[...], w_ref[...], preferred_element_type=jnp.float32)
  o_ref[...] = jnp.dot(h, a_ref[...], preferred_element_type=jnp.float32)


def _tc_attdst(x, w, a16):
  n = x.shape[0]
  br = 1000
  return pl.pallas_call(
      _attdst_body,
      grid=(n // br,),
      in_specs=[
          pl.BlockSpec((br, HID), lambda i: (i, 0)),
          pl.BlockSpec((HID, HID), lambda i: (0, 0)),
          pl.BlockSpec((HID, 16), lambda i: (0, 0)),
      ],
      out_specs=pl.BlockSpec((br, 16), lambda i: (i, 0)),
      out_shape=jax.ShapeDtypeStruct((n, 16), jnp.float32),
  )(x, w, a16)


def _denc_body(p_ref, o_ref):
  o_ref[...] = p_ref[0] + p_ref[1] + 1e-16


def _tc_den_combine(p):
  nd8 = p.shape[2]
  return pl.pallas_call(
      _denc_body,
      grid=(1,),
      in_specs=[pl.BlockSpec((2, HEADS, nd8), lambda i: (0, 0, 0))],
      out_specs=pl.BlockSpec((HEADS, nd8), lambda i: (0, 0)),
      out_shape=jax.ShapeDtypeStruct((HEADS, nd8), jnp.float32),
  )(p)


def _comb_c_body(xc_ref, p_ref, b_ref, w_ref, a_ref, hc_ref, hs_ref, a16_ref):
  hc = xc_ref[...] + 0.5 * (p_ref[0] + p_ref[1] + b_ref[...])
  hc_ref[...] = hc
  hs = jnp.dot(hc, w_ref[...], preferred_element_type=jnp.float32)
  hs_ref[...] = hs
  a16_ref[...] = jnp.dot(hs, a_ref[...], preferred_element_type=jnp.float32)


def _tc_combine_c(xc, p, bias, wsrc, a_s):
  br = 1000
  return pl.pallas_call(
      _comb_c_body,
      grid=(NC // br,),
      in_specs=[
          pl.BlockSpec((br, HID), lambda i: (i, 0)),
          pl.BlockSpec((2, br, HID), lambda i: (0, i, 0)),
          pl.BlockSpec((1, HID), lambda i: (0, 0)),
          pl.BlockSpec((HID, HID), lambda i: (0, 0)),
          pl.BlockSpec((HID, 16), lambda i: (0, 0)),
      ],
      out_specs=[
          pl.BlockSpec((br, HID), lambda i: (i, 0)),
          pl.BlockSpec((br, HID), lambda i: (i, 0)),
          pl.BlockSpec((br, 16), lambda i: (i, 0)),
      ],
      out_shape=[
          jax.ShapeDtypeStruct((NC, HID), jnp.float32),
          jax.ShapeDtypeStruct((NC, HID), jnp.float32),
          jax.ShapeDtypeStruct((NC, 16), jnp.float32),
      ],
  )(xc, p, bias, wsrc, a_s)


def _final_t_body(xt_ref, q_ref, b_ref, o_ref):
  o_ref[...] = xt_ref[...] + 0.5 * (q_ref[0] + q_ref[1] + b_ref[...])


def _tc_final_t(xt, q, bias):
  return pl.pallas_call(
      _final_t_body,
      grid=(1,),
      in_specs=[
          pl.BlockSpec((NT, HID), lambda i: (0, 0)),
          pl.BlockSpec((2, NT, HID), lambda i: (0, 0, 0)),
          pl.BlockSpec((1, HID), lambda i: (0, 0)),
      ],
      out_specs=pl.BlockSpec((NT, HID), lambda i: (0, 0)),
      out_shape=jax.ShapeDtypeStruct((NT, HID), jnp.float32),
  )(xt, q, bias)


def _final_b_body(xb_ref, xln_ref, dis_ref, g_ref, bg_ref, o_ref):
  g = g_ref[0, 0] + g_ref[1, 0]
  o_ref[...] = xb_ref[...] + 0.2 * (
      dis_ref[...] * (g + xln_ref[...]) + bg_ref[...])


def _tc_final_b(xb, xln, dis, g, bg):
  br = 3576
  nj = CHUNK // br
  return pl.pallas_call(
      _final_b_body,
      grid=(NPASS, nj),
      in_specs=[
          pl.BlockSpec((br, HID), lambda i, j: (i * nj + j, 0)),
          pl.BlockSpec((br, HID), lambda i, j: (i * nj + j, 0)),
          pl.BlockSpec((br, 1), lambda i, j: (i * nj + j, 0)),
          pl.BlockSpec((2, 1, br, HID), lambda i, j: (0, i, j, 0)),
          pl.BlockSpec((1, HID), lambda i, j: (0, 0)),
      ],
      out_specs=pl.BlockSpec((br, HID), lambda i, j: (i * nj + j, 0)),
      out_shape=jax.ShapeDtypeStruct((NB, HID), jnp.float32),
  )(xb, xln, dis, g, bg)


def _mesh():
  return plsc.VectorSubcoreMesh(core_axis_name="c", subcore_axis_name="s")


def _zero_vec_buf(ref, n):
  z = jnp.zeros((16,), jnp.float32)
  def body(i, _):
    ref[pl.ds(i * 16, 16)] = z
    return 0
  lax.fori_loop(0, n // 16, body, 0)


def _zero_rows_buf(ref, rows, width=128):
  z = jnp.zeros((16,), jnp.float32)
  w = width // 16
  def body(i, _):
    r = i // w
    k = i % w
    ref[r, pl.ds(k * 16, 16)] = z
    return 0
  lax.fori_loop(0, rows * w, body, 0)


def _zero_share_rows(zsrc, dst, base, share):
  full, rem = divmod(share, 128)
  for t in range(full):
    pltpu.sync_copy(zsrc, dst.at[pl.ds(base + t * 128, 128)])
  if rem:
    pltpu.sync_copy(zsrc.at[pl.ds(0, rem)], dst.at[pl.ds(base + full * 128, rem)])


def _zero_share_1d(zbuf, sp, base, share):
  full, rem = divmod(share, 2048)
  for t in range(full):
    pltpu.sync_copy(zbuf, sp.at[pl.ds(base + t * 2048, 2048)])
  if rem:
    pltpu.sync_copy(zbuf.at[pl.ds(0, rem)],
                    sp.at[pl.ds(base + full * 2048, rem)])


def _sp_to_hbm(sp_ref, out_slice, bounce, base, share, brows):
  full, rem = divmod(share, brows)
  for t in range(full):
    o = base + t * brows
    pltpu.sync_copy(sp_ref.at[pl.ds(o, brows)], bounce)
    pltpu.sync_copy(bounce, out_slice(o, brows))
  if rem:
    o = base + full * brows
    pltpu.sync_copy(sp_ref.at[pl.ds(o, rem)], bounce.at[pl.ds(0, rem)])
    pltpu.sync_copy(bounce.at[pl.ds(0, rem)], out_slice(o, rem))


def _sc_deg(dst_pad):
  @functools.partial(
      pl.kernel,
      out_type=jax.ShapeDtypeStruct((2 * DEG_N,), jnp.float32),
      mesh=_mesh(),
      compiler_params=pltpu.CompilerParams(needs_layout_passes=False),
      scratch_types=[
          pltpu.VMEM((EPT2,), jnp.int32),
          pltpu.VMEM((128,), jnp.float32),
          pltpu.VMEM((128,), jnp.int32),
          pltpu.VMEM((2048,), jnp.float32),
          pltpu.VMEM_SHARED((DEG_N,), jnp.float32),
      ],
  )
  def k(dst_hbm, deg_out, dbuf, ones_v, didx, zbuf, deg_sp):
    c = lax.axis_index("c")
    s = lax.axis_index("s")
    wid = s * NCORES + c
    pltpu.sync_copy(dst_hbm.at[pl.ds(wid * EPT2, EPT2)], dbuf)
    _zero_vec_buf(zbuf, 2048)
    one = jnp.ones((16,), jnp.float32)
    def ob(i, _):
      ones_v[pl.ds(i * 16, 16)] = one
      return 0
    lax.fori_loop(0, 8, ob, 0)
    base = s * SHARE_D
    for t in range(SHARE_D // 2048):
      pltpu.sync_copy(zbuf, deg_sp.at[pl.ds(base + t * 2048, 2048)])
    rem = SHARE_D % 2048
    if rem:
      pltpu.sync_copy(zbuf.at[pl.ds(0, rem)],
                      deg_sp.at[pl.ds(base + (SHARE_D // 2048) * 2048, rem)])
    plsc.subcore_barrier()
    def blk(j, _):
      def cp(kk, _):
        didx[pl.ds(kk * 16, 16)] = dbuf[pl.ds(j * 128 + kk * 16, 16)]
        return 0
      lax.fori_loop(0, 8, cp, 0)
      pltpu.sync_copy(ones_v, deg_sp.at[didx], add=True)
      return 0
    lax.fori_loop(0, EPT2 // 128, blk, 0)
    plsc.subcore_barrier()
    _sp_to_hbm(deg_sp, lambda o, n: deg_out.at[pl.ds(c * DEG_N + o, n)],
               zbuf, base, SHARE_D, 2048)

  return k(dst_pad)


def _sc_gat_den(src_pad, dst_pad, asrc_h, adst_h, epad, ept, nblk, nd8):
  share = nd8 // NSUB

  @functools.partial(
      pl.kernel,
      out_type=(
          jax.ShapeDtypeStruct((HEADS * epad,), jnp.float32),
          jax.ShapeDtypeStruct((2 * HEADS * nd8,), jnp.float32),
      ),
      mesh=_mesh(),
      compiler_params=pltpu.CompilerParams(needs_layout_passes=False),
      scratch_types=[
          pltpu.VMEM((128,), jnp.int32),
          pltpu.VMEM((128,), jnp.int32),
          pltpu.VMEM((HEADS, 128), jnp.float32),
          pltpu.VMEM((HEADS, 128), jnp.float32),
          pltpu.VMEM((HEADS, 128), jnp.float32),
          pltpu.VMEM((128,), jnp.int32),
          pltpu.VMEM((2048,), jnp.float32),
          pltpu.VMEM_SHARED((HEADS * nd8,), jnp.float32),
          pltpu.SemaphoreType.DMA,
      ],
  )
  def k(src_hbm, dst_hbm, as0, as1, as2, as3, ad0, ad1, ad2, ad3,
        ex_out, den_out, sidx, didx, asg, adg, exb, didxo, zbuf, den_sp,
        gsem):
    asrc = (as0, as1, as2, as3)
    adst = (ad0, ad1, ad2, ad3)
    c = lax.axis_index("c")
    s = lax.axis_index("s")
    wid = s * NCORES + c
    _zero_vec_buf(zbuf, 2048)
    share2 = HEADS * share
    base2 = s * share2
    _zero_share_1d(zbuf, den_sp, base2, share2)
    plsc.subcore_barrier()

    def blk(j, _):
      e0 = wid * ept + j * 128
      pltpu.sync_copy(src_hbm.at[pl.ds(e0, 128)], sidx)
      pltpu.sync_copy(dst_hbm.at[pl.ds(e0, 128)], didx)
      ds_list = []
      for h in range(HEADS):
        ds_list.append(pltpu.async_copy(asrc[h].at[sidx], asg.at[h], gsem))
        ds_list.append(pltpu.async_copy(adst[h].at[didx], adg.at[h], gsem))
      for d in ds_list:
        d.wait()
      for h in range(HEADS):
        for kk in range(8):
          sl = pl.ds(kk * 16, 16)
          al = asg[h, sl] + adg[h, sl]
          al = jnp.where(al >= 0, al, 0.2 * al)
          exb[h, sl] = jnp.exp(al)
      for h in range(HEADS):
        pltpu.sync_copy(exb.at[h], ex_out.at[pl.ds(h * epad + e0, 128)])
        def off(kk, _, h=h):
          sl = pl.ds(kk * 16, 16)
          didxo[sl] = didx[sl] + h * nd8
          return 0
        lax.fori_loop(0, 8, off, 0)
        pltpu.sync_copy(exb.at[h], den_sp.at[didxo], add=True)
      return 0
    lax.fori_loop(0, nblk, blk, 0)
    plsc.subcore_barrier()
    _sp_to_hbm(den_sp,
               lambda o, n: den_out.at[pl.ds(c * HEADS * nd8 + o, n)],
               zbuf, base2, share2, 2048)

  return k(src_pad, dst_pad, *asrc_h, *adst_h)


def _sc_gat_agg(src_pad, dst_pad, ex, den_h, hs, epad, ept, nblk, nd8):
  share = nd8 // NSUB

  @functools.partial(
      pl.kernel,
      out_type=jax.ShapeDtypeStruct((2, nd8, HID), jnp.float32),
      mesh=_mesh(),
      compiler_params=pltpu.CompilerParams(needs_layout_passes=False),
      scratch_types=[
          pltpu.VMEM((128,), jnp.int32),
          pltpu.VMEM((128,), jnp.int32),
          pltpu.VMEM((HEADS, 128), jnp.float32),
          pltpu.VMEM((HEADS, 128), jnp.float32),
          pltpu.VMEM((HEADS * 128,), jnp.float32),
          pltpu.VMEM((128, 128), jnp.float32),
          pltpu.VMEM((128, 128), jnp.float32),
          pltpu.VMEM_SHARED((nd8, HID), jnp.float32),
          pltpu.SemaphoreType.DMA,
      ],
  )
  def k(src_hbm, dst_hbm, ex_hbm, dn0, dn1, dn2, dn3, hs_hbm, out_hbm,
        sidx, didx, exb, denb, attnT, rows, zrows, out_sp, gsem):
    den = (dn0, dn1, dn2, dn3)
    c = lax.axis_index("c")
    s = lax.axis_index("s")
    wid = s * NCORES + c
    _zero_rows_buf(zrows, 128)
    base = s * share
    _zero_share_rows(zrows, out_sp, base, share)
    plsc.subcore_barrier()
    iota16 = lax.iota(jnp.int32, 16)

    def blk(j, _):
      e0 = wid * ept + j * 128
      pltpu.sync_copy(src_hbm.at[pl.ds(e0, 128)], sidx)
      pltpu.sync_copy(dst_hbm.at[pl.ds(e0, 128)], didx)
      ds_list = [pltpu.async_copy(hs_hbm.at[sidx], rows, gsem)]
      for h in range(HEADS):
        ds_list.append(pltpu.async_copy(den[h].at[didx], denb.at[h], gsem))
        pltpu.sync_copy(ex_hbm.at[pl.ds(h * epad + e0, 128)], exb.at[h])
      for d in ds_list:
        d.wait()
      for h in range(HEADS):
        for kk in range(8):
          sl = pl.ds(kk * 16, 16)
          attnT[pl.ds(h * 128 + kk * 16, 16)] = exb[h, sl] / denb[h, sl]
      def grp(g, _):
        avs = [attnT[pl.ds(h * 128 + g * 16, 16)] for h in range(HEADS)]
        def rw(l, _):
          i = g * 16 + l
          onehot = (iota16 == jnp.broadcast_to(l, (16,))).astype(jnp.float32)
          for h in range(HEADS):
            scv = jnp.broadcast_to(jnp.sum(avs[h] * onehot), (16,))
            for kk in range(2):
              c0 = h * 32 + kk * 16
              rows[i, pl.ds(c0, 16)] = rows[i, pl.ds(c0, 16)] * scv
          return 0
        lax.fori_loop(0, 16, rw, 0)
        return 0
      lax.fori_loop(0, 8, grp, 0)
      pltpu.sync_copy(rows, out_sp.at[didx], add=True)
      return 0
    lax.fori_loop(0, nblk, blk, 0)
    plsc.subcore_barrier()
    _sp_to_hbm(out_sp, lambda o, n: out_hbm.at[c, pl.ds(o, n)],
               rows, base, share, 128)

  return k(src_pad, dst_pad, ex, *den_h, hs)


def _sc_gcn(src_pad, dst_pad, xln):
  @functools.partial(
      pl.kernel,
      out_type=jax.ShapeDtypeStruct((2, NPASS, CHR, HID), jnp.float32),
      mesh=_mesh(),
      compiler_params=pltpu.CompilerParams(needs_layout_passes=False),
      scratch_types=[
          pltpu.VMEM((EPT2,), jnp.int32),
          pltpu.VMEM((EPT2,), jnp.int32),
          pltpu.VMEM((CPK_CAP + 16,), jnp.int32),
          pltpu.VMEM((128,), jnp.int32),
          pltpu.VMEM((128,), jnp.int32),
          pltpu.VMEM((128, 128), jnp.float32),
          pltpu.VMEM_SHARED((CHR, HID), jnp.float32),
          pltpu.SemaphoreType.DMA,
      ],
  )
  def k(src_hbm, dst_hbm, xln_hbm, g_out,
        sbuf, dbuf, cpk, sidx, didx, rows, chunk, gsem):
    c = lax.axis_index("c")
    s = lax.axis_index("s")
    wid = s * NCORES + c
    pltpu.sync_copy(src_hbm.at[pl.ds(wid * EPT2, EPT2)], sbuf)
    pltpu.sync_copy(dst_hbm.at[pl.ds(wid * EPT2, EPT2)], dbuf)
    base = s * SHARE_G
    iota = lax.iota(jnp.int32, 16)
    trash_pk = GTRASH * 131072

    def do_pass(p, _):
      lov = jnp.broadcast_to(p * CHUNK, (16,))
      hiv = lov + CHUNK
      _zero_rows_buf(rows, 128)
      _zero_share_rows(rows, chunk, base, SHARE_G)
      plsc.subcore_barrier()

      def cvec(v, cnt):
        off = v * 16
        srcv = sbuf[pl.ds(off, 16)]
        dstv = dbuf[pl.ds(off, 16)]
        m = (dstv >= lov) & (dstv < hiv)
        packed = (dstv - lov) * 131072 + srcv
        mi = m.astype(jnp.int32)
        pos = jnp.broadcast_to(cnt, (16,)) + plsc.cumsum(mi) - 1
        pos = jnp.where(m, pos, CPK_CAP + iota)
        plsc.store_scatter(cpk, [pos], packed)
        return cnt + jnp.sum(mi)
      cnt = lax.fori_loop(0, NV2, cvec, jnp.int32(0))

      cntv = jnp.broadcast_to(cnt, (16,))
      for t in range(8):
        plsc.store_scatter(cpk, [cntv + iota + t * 16],
                           jnp.full((16,), trash_pk, jnp.int32))
      nblk = (cnt + 127) // 128

      def gs(j, _):
        def up(kk, _):
          pv = cpk[pl.ds(j * 128 + kk * 16, 16)]
          sidx[pl.ds(kk * 16, 16)] = pv & 131071
          didx[pl.ds(kk * 16, 16)] = lax.shift_right_logical(pv, 17)
          return 0
        lax.fori_loop(0, 8, up, 0)
        pltpu.async_copy(xln_hbm.at[sidx], rows, gsem).wait()
        pltpu.sync_copy(rows, chunk.at[didx], add=True)
        return 0
      lax.fori_loop(0, nblk, gs, 0)
      plsc.subcore_barrier()
      _sp_to_hbm(chunk, lambda o, n: g_out.at[c, p, pl.ds(o, n)],
                 rows, base, SHARE_G, 128)
      plsc.subcore_barrier()
      return 0
    lax.fori_loop(0, NPASS, do_pass, 0)

  return k(src_pad, dst_pad, xln)


def _att_mat(att):
  r = jnp.arange(HID)
  return jnp.zeros((HID, 16), jnp.float32).at[r, r // CH].set(att.reshape(-1))


def _pad_edges(ei, epad, trash_dst):
  e = ei.shape[1]
  src = jnp.concatenate([ei[0], jnp.zeros((epad - e,), jnp.int32)])
  dst = jnp.concatenate([ei[1], jnp.full((epad - e,), trash_dst, jnp.int32)])
  return src, dst


def kernel(x_building, x_cable_group, x_transformer, edge_index_b2c,
           edge_index_c2t, edge_index_b2b, W_src_bl, W_dst_bl, att_src_bl,
           att_dst_bl, bias_bl, W_src_lt, W_dst_lt, att_src_lt, att_dst_lt,
           bias_lt, W_gcn, b_gcn):
  src1, dst1 = _pad_edges(edge_index_b2c, EPAD1, NC)
  src3, dst3 = _pad_edges(edge_index_c2t, EPAD3, NT)
  src2, dst2 = _pad_edges(edge_index_b2b, EPAD2, NB)

  degp = _sc_deg(dst2).reshape(2, DEG_N)
  deg = degp[0, :NB] + degp[1, :NB] + 1.0
  dis = lax.rsqrt(deg).reshape(NB, 1)

  a_s_bl = _att_mat(att_src_bl)
  a_d_bl = _att_mat(att_dst_bl)
  hs_b, asrc16_b, xln = _tc_prep_b(x_building, dis, W_src_bl, a_s_bl, W_gcn)

  adst16_c = _tc_attdst(x_cable_group, W_dst_bl, a_d_bl)
  adst16_c = jnp.concatenate(
      [adst16_c, jnp.zeros((ND8_C - NC, 16), jnp.float32)])
  asrc_h_b = tuple(asrc16_b[:, h] for h in range(HEADS))
  adst_h_c = tuple(adst16_c[:, h] for h in range(HEADS))

  ex1, denp1 = _sc_gat_den(src1, dst1, asrc_h_b, adst_h_c,
                           EPAD1, EPT1, NBLK1, ND8_C)
  den1 = _tc_den_combine(denp1.reshape(2, HEADS, ND8_C))
  den1_h = tuple(den1[h] for h in range(HEADS))
  outc_p = _sc_gat_agg(src1, dst1, ex1, den1_h, hs_b,
                       EPAD1, EPT1, NBLK1, ND8_C)
  h_c, hs_c, asrc16_c = _tc_combine_c(
      x_cable_group, outc_p[:, :NC], bias_bl.reshape(1, HID), W_src_lt,
      _att_mat(att_src_lt))

  adst16_t = _tc_attdst(x_transformer, W_dst_lt, _att_mat(att_dst_lt))
  adst16_t = jnp.concatenate(
      [adst16_t, jnp.zeros((ND8_T - NT, 16), jnp.float32)])
  asrc_h_c = tuple(asrc16_c[:, h] for h in range(HEADS))
  adst_h_t = tuple(adst16_t[:, h] for h in range(HEADS))
  ex3, denp3 = _sc_gat_den(src3, dst3, asrc_h_c, adst_h_t,
                           EPAD3, EPT3, NBLK3, ND8_T)
  den3 = _tc_den_combine(denp3.reshape(2, HEADS, ND8_T))
  den3_h = tuple(den3[h] for h in range(HEADS))
  outt_p = _sc_gat_agg(src3, dst3, ex3, den3_h, hs_c,
                       EPAD3, EPT3, NBLK3, ND8_T)
  h_t = _tc_final_t(x_transformer, outt_p, bias_lt.reshape(1, HID))

  g = _sc_gcn(src2, dst2, xln)
  h_b = _tc_final_b(x_building, xln, dis, g, b_gcn.reshape(1, HID))

  return (h_b, h_c, h_t)


# speedup vs baseline: 11.1693x; 1.0028x over previous
import functools

import jax
import jax.numpy as jnp
from jax import lax
from jax.experimental import pallas as pl
from jax.experimental.pallas import tpu as pltpu
from jax.experimental.pallas import tpu_sc as plsc

HID = 128
HEADS = 4
CH = 32
NB = 100000
NC = 10000
NT = 1000

NCORES = 2
NSUB = 16
NTILES = NCORES * NSUB

E1 = 100000
EPT1 = 3200
NBLK1 = EPT1 // 128
EPAD1 = EPT1 * NTILES
ND8_C = 10112
SHARE_C = ND8_C // NSUB

E3 = 10000
EPT3 = 384
NBLK3 = EPT3 // 128
EPAD3 = EPT3 * NTILES
ND8_T = 1024
SHARE_T = ND8_T // NSUB

E2 = 600000
EPT2 = 18816
NV2 = EPT2 // 16
EPAD2 = EPT2 * NTILES
NPASS = 14
CHUNK = 7152
CHR = 7168
GTRASH = 7152
SHARE_G = CHR // NSUB
CPK_CAP = EPT2 + 128

DEG_N = 100096
SHARE_D = DEG_N // NSUB


def _prep_b_body(xb_ref, dis_ref, ws_ref, as_ref, wg_ref, hs_ref, a16_ref,
                 xln_ref):
  xb = xb_ref[...]
  hs = jnp.dot(xb, ws_ref[...], preferred_element_type=jnp.float32)
  hs_ref[...] = hs
  a16_ref[...] = jnp.dot(hs, as_ref[...], preferred_element_type=jnp.float32)
  xl = jnp.dot(xb, wg_ref[...], preferred_element_type=jnp.float32)
  xln_ref[...] = xl * dis_ref[...]


def _tc_prep_b(xb, dis, wsrc, a_s, wgcn):
  br = 1000
  return pl.pallas_call(
      _prep_b_body,
      grid=(NB // br,),
      in_specs=[
          pl.BlockSpec((br, HID), lambda i: (i, 0)),
          pl.BlockSpec((br, 1), lambda i: (i, 0)),
          pl.BlockSpec((HID, HID), lambda i: (0, 0)),
          pl.BlockSpec((HID, 16), lambda i: (0, 0)),
          pl.BlockSpec((HID, HID), lambda i: (0, 0)),
      ],
      out_specs=[
          pl.BlockSpec((br, HID), lambda i: (i, 0)),
          pl.BlockSpec((br, 16), lambda i: (i, 0)),
          pl.BlockSpec((br, HID), lambda i: (i, 0)),
      ],
      out_shape=[
          jax.ShapeDtypeStruct((NB, HID), jnp.float32),
          jax.ShapeDtypeStruct((NB, 16), jnp.float32),
          jax.ShapeDtypeStruct((NB, HID), jnp.float32),
      ],
  )(xb, dis, wsrc, a_s, wgcn)


def _attdst_body(x_ref, w_ref, a_ref, o_ref):
  h = jnp.dot(x_ref[...], w_ref[...], preferred_element_type=jnp.float32)
  o_ref[...] = jnp.dot(h, a_ref[...], preferred_element_type=jnp.float32)


def _tc_attdst(x, w, a16):
  n = x.shape[0]
  br = 1000
  return pl.pallas_call(
      _attdst_body,
      grid=(n // br,),
      in_specs=[
          pl.BlockSpec((br, HID), lambda i: (i, 0)),
          pl.BlockSpec((HID, HID), lambda i: (0, 0)),
          pl.BlockSpec((HID, 16), lambda i: (0, 0)),
      ],
      out_specs=pl.BlockSpec((br, 16), lambda i: (i, 0)),
      out_shape=jax.ShapeDtypeStruct((n, 16), jnp.float32),
  )(x, w, a16)


def _denc_body(p_ref, o_ref):
  o_ref[...] = p_ref[0] + p_ref[1] + 1e-16


def _tc_den_combine(p):
  nd8 = p.shape[2]
  return pl.pallas_call(
      _denc_body,
      grid=(1,),
      in_specs=[pl.BlockSpec((2, HEADS, nd8), lambda i: (0, 0, 0))],
      out_specs=pl.BlockSpec((HEADS, nd8), lambda i: (0, 0)),
      out_shape=jax.ShapeDtypeStruct((HEADS, nd8), jnp.float32),
  )(p)


def _comb_c_body(xc_ref, p_ref, b_ref, w_ref, a_ref, hc_ref, hs_ref, a16_ref):
  hc = xc_ref[...] + 0.5 * (p_ref[0] + p_ref[1] + b_ref[...])
  hc_ref[...] = hc
  hs = jnp.dot(hc, w_ref[...], preferred_element_type=jnp.float32)
  hs_ref[...] = hs
  a16_ref[...] = jnp.dot(hs, a_ref[...], preferred_element_type=jnp.float32)


def _tc_combine_c(xc, p, bias, wsrc, a_s):
  br = 1000
  return pl.pallas_call(
      _comb_c_body,
      grid=(NC // br,),
      in_specs=[
          pl.BlockSpec((br, HID), lambda i: (i, 0)),
          pl.BlockSpec((2, br, HID), lambda i: (0, i, 0)),
          pl.BlockSpec((1, HID), lambda i: (0, 0)),
          pl.BlockSpec((HID, HID), lambda i: (0, 0)),
          pl.BlockSpec((HID, 16), lambda i: (0, 0)),
      ],
      out_specs=[
          pl.BlockSpec((br, HID), lambda i: (i, 0)),
          pl.BlockSpec((br, HID), lambda i: (i, 0)),
          pl.BlockSpec((br, 16), lambda i: (i, 0)),
      ],
      out_shape=[
          jax.ShapeDtypeStruct((NC, HID), jnp.float32),
          jax.ShapeDtypeStruct((NC, HID), jnp.float32),
          jax.ShapeDtypeStruct((NC, 16), jnp.float32),
      ],
  )(xc, p, bias, wsrc, a_s)


def _final_t_body(xt_ref, q_ref, b_ref, o_ref):
  o_ref[...] = xt_ref[...] + 0.5 * (q_ref[0] + q_ref[1] + b_ref[...])


def _tc_final_t(xt, q, bias):
  return pl.pallas_call(
      _final_t_body,
      grid=(1,),
      in_specs=[
          pl.BlockSpec((NT, HID), lambda i: (0, 0)),
          pl.BlockSpec((2, NT, HID), lambda i: (0, 0, 0)),
          pl.BlockSpec((1, HID), lambda i: (0, 0)),
      ],
      out_specs=pl.BlockSpec((NT, HID), lambda i: (0, 0)),
      out_shape=jax.ShapeDtypeStruct((NT, HID), jnp.float32),
  )(xt, q, bias)


def _final_b_body(xb_ref, xln_ref, dis_ref, g_ref, bg_ref, o_ref):
  g = g_ref[0, 0] + g_ref[1, 0]
  o_ref[...] = xb_ref[...] + 0.2 * (
      dis_ref[...] * (g + xln_ref[...]) + bg_ref[...])


def _tc_final_b(xb, xln, dis, g, bg):
  br = 3576
  nj = CHUNK // br
  return pl.pallas_call(
      _final_b_body,
      grid=(NPASS, nj),
      in_specs=[
          pl.BlockSpec((br, HID), lambda i, j: (i * nj + j, 0)),
          pl.BlockSpec((br, HID), lambda i, j: (i * nj + j, 0)),
          pl.BlockSpec((br, 1), lambda i, j: (i * nj + j, 0)),
          pl.BlockSpec((2, 1, br, HID), lambda i, j: (0, i, j, 0)),
          pl.BlockSpec((1, HID), lambda i, j: (0, 0)),
      ],
      out_specs=pl.BlockSpec((br, HID), lambda i, j: (i * nj + j, 0)),
      out_shape=jax.ShapeDtypeStruct((NB, HID), jnp.float32),
  )(xb, xln, dis, g, bg)


def _mesh():
  return plsc.VectorSubcoreMesh(core_axis_name="c", subcore_axis_name="s")


def _zero_vec_buf(ref, n):
  z = jnp.zeros((16,), jnp.float32)
  def body(i, _):
    ref[pl.ds(i * 16, 16)] = z
    return 0
  lax.fori_loop(0, n // 16, body, 0)


def _zero_rows_buf(ref, rows, width=128):
  z = jnp.zeros((16,), jnp.float32)
  w = width // 16
  def body(i, _):
    r = i // w
    k = i % w
    ref[r, pl.ds(k * 16, 16)] = z
    return 0
  lax.fori_loop(0, rows * w, body, 0)


def _zero_share_rows(zsrc, dst, base, share):
  full, rem = divmod(share, 128)
  for t in range(full):
    pltpu.sync_copy(zsrc, dst.at[pl.ds(base + t * 128, 128)])
  if rem:
    pltpu.sync_copy(zsrc.at[pl.ds(0, rem)], dst.at[pl.ds(base + full * 128, rem)])


def _zero_share_1d(zbuf, sp, base, share):
  full, rem = divmod(share, 2048)
  for t in range(full):
    pltpu.sync_copy(zbuf, sp.at[pl.ds(base + t * 2048, 2048)])
  if rem:
    pltpu.sync_copy(zbuf.at[pl.ds(0, rem)],
                    sp.at[pl.ds(base + full * 2048, rem)])


def _sp_to_hbm(sp_ref, out_slice, bounce, base, share, brows):
  full, rem = divmod(share, brows)
  for t in range(full):
    o = base + t * brows
    pltpu.sync_copy(sp_ref.at[pl.ds(o, brows)], bounce)
    pltpu.sync_copy(bounce, out_slice(o, brows))
  if rem:
    o = base + full * brows
    pltpu.sync_copy(sp_ref.at[pl.ds(o, rem)], bounce.at[pl.ds(0, rem)])
    pltpu.sync_copy(bounce.at[pl.ds(0, rem)], out_slice(o, rem))


def _sc_deg(dst_pad):
  @functools.partial(
      pl.kernel,
      out_type=jax.ShapeDtypeStruct((2 * DEG_N,), jnp.float32),
      mesh=_mesh(),
      compiler_params=pltpu.CompilerParams(needs_layout_passes=False),
      scratch_types=[
          pltpu.VMEM((EPT2,), jnp.int32),
          pltpu.VMEM((128,), jnp.float32),
          pltpu.VMEM((128,), jnp.int32),
          pltpu.VMEM((2048,), jnp.float32),
          pltpu.VMEM_SHARED((DEG_N,), jnp.float32),
      ],
  )
  def k(dst_hbm, deg_out, dbuf, ones_v, didx, zbuf, deg_sp):
    c = lax.axis_index("c")
    s = lax.axis_index("s")
    wid = s * NCORES + c
    pltpu.sync_copy(dst_hbm.at[pl.ds(wid * EPT2, EPT2)], dbuf)
    _zero_vec_buf(zbuf, 2048)
    one = jnp.ones((16,), jnp.float32)
    def ob(i, _):
      ones_v[pl.ds(i * 16, 16)] = one
      return 0
    lax.fori_loop(0, 8, ob, 0)
    base = s * SHARE_D
    for t in range(SHARE_D // 2048):
      pltpu.sync_copy(zbuf, deg_sp.at[pl.ds(base + t * 2048, 2048)])
    rem = SHARE_D % 2048
    if rem:
      pltpu.sync_copy(zbuf.at[pl.ds(0, rem)],
                      deg_sp.at[pl.ds(base + (SHARE_D // 2048) * 2048, rem)])
    plsc.subcore_barrier()
    def blk(j, _):
      def cp(kk, _):
        didx[pl.ds(kk * 16, 16)] = dbuf[pl.ds(j * 128 + kk * 16, 16)]
        return 0
      lax.fori_loop(0, 8, cp, 0)
      pltpu.sync_copy(ones_v, deg_sp.at[didx], add=True)
      return 0
    lax.fori_loop(0, EPT2 // 128, blk, 0)
    plsc.subcore_barrier()
    _sp_to_hbm(deg_sp, lambda o, n: deg_out.at[pl.ds(c * DEG_N + o, n)],
               zbuf, base, SHARE_D, 2048)

  return k(dst_pad)


def _sc_gat_den(src_pad, dst_pad, asrc_h, adst_h, epad, ept, nblk, nd8):
  share = nd8 // NSUB

  @functools.partial(
      pl.kernel,
      out_type=(
          jax.ShapeDtypeStruct((HEADS * epad,), jnp.float32),
          jax.ShapeDtypeStruct((2 * HEADS * nd8,), jnp.float32),
      ),
      mesh=_mesh(),
      compiler_params=pltpu.CompilerParams(needs_layout_passes=False),
      scratch_types=[
          pltpu.VMEM((128,), jnp.int32),
          pltpu.VMEM((128,), jnp.int32),
          pltpu.VMEM((HEADS, 128), jnp.float32),
          pltpu.VMEM((HEADS, 128), jnp.float32),
          pltpu.VMEM((HEADS, 128), jnp.float32),
          pltpu.VMEM((128,), jnp.int32),
          pltpu.VMEM((2048,), jnp.float32),
          pltpu.VMEM_SHARED((HEADS * nd8,), jnp.float32),
          pltpu.SemaphoreType.DMA,
      ],
  )
  def k(src_hbm, dst_hbm, as0, as1, as2, as3, ad0, ad1, ad2, ad3,
        ex_out, den_out, sidx, didx, asg, adg, exb, didxo, zbuf, den_sp,
        gsem):
    asrc = (as0, as1, as2, as3)
    adst = (ad0, ad1, ad2, ad3)
    c = lax.axis_index("c")
    s = lax.axis_index("s")
    wid = s * NCORES + c
    _zero_vec_buf(zbuf, 2048)
    share2 = HEADS * share
    base2 = s * share2
    _zero_share_1d(zbuf, den_sp, base2, share2)
    plsc.subcore_barrier()

    def blk(j, _):
      e0 = wid * ept + j * 128
      pltpu.sync_copy(src_hbm.at[pl.ds(e0, 128)], sidx)
      pltpu.sync_copy(dst_hbm.at[pl.ds(e0, 128)], didx)
      ds_list = []
      for h in range(HEADS):
        ds_list.append(pltpu.async_copy(asrc[h].at[sidx], asg.at[h], gsem))
        ds_list.append(pltpu.async_copy(adst[h].at[didx], adg.at[h], gsem))
      for d in ds_list:
        d.wait()
      for h in range(HEADS):
        for kk in range(8):
          sl = pl.ds(kk * 16, 16)
          al = asg[h, sl] + adg[h, sl]
          al = jnp.where(al >= 0, al, 0.2 * al)
          exb[h, sl] = jnp.exp(al)
      for h in range(HEADS):
        pltpu.sync_copy(exb.at[h], ex_out.at[pl.ds(h * epad + e0, 128)])
        def off(kk, _, h=h):
          sl = pl.ds(kk * 16, 16)
          didxo[sl] = didx[sl] + h * nd8
          return 0
        lax.fori_loop(0, 8, off, 0)
        pltpu.sync_copy(exb.at[h], den_sp.at[didxo], add=True)
      return 0
    lax.fori_loop(0, nblk, blk, 0)
    plsc.subcore_barrier()
    _sp_to_hbm(den_sp,
               lambda o, n: den_out.at[pl.ds(c * HEADS * nd8 + o, n)],
               zbuf, base2, share2, 2048)

  return k(src_pad, dst_pad, *asrc_h, *adst_h)


def _sc_gat_agg(src_pad, dst_pad, ex, den_h, hs, epad, ept, nblk, nd8):
  share = nd8 // NSUB

  @functools.partial(
      pl.kernel,
      out_type=jax.ShapeDtypeStruct((2, nd8, HID), jnp.float32),
      mesh=_mesh(),
      compiler_params=pltpu.CompilerParams(needs_layout_passes=False),
      scratch_types=[
          pltpu.VMEM((128,), jnp.int32),
          pltpu.VMEM((128,), jnp.int32),
          pltpu.VMEM((HEADS, 128), jnp.float32),
          pltpu.VMEM((HEADS, 128), jnp.float32),
          pltpu.VMEM((HEADS * 128,), jnp.float32),
          pltpu.VMEM((128, 128), jnp.float32),
          pltpu.VMEM((128, 128), jnp.float32),
          pltpu.VMEM_SHARED((nd8, HID), jnp.float32),
          pltpu.SemaphoreType.DMA,
      ],
  )
  def k(src_hbm, dst_hbm, ex_hbm, dn0, dn1, dn2, dn3, hs_hbm, out_hbm,
        sidx, didx, exb, denb, attnT, rows, zrows, out_sp, gsem):
    den = (dn0, dn1, dn2, dn3)
    c = lax.axis_index("c")
    s = lax.axis_index("s")
    wid = s * NCORES + c
    _zero_rows_buf(zrows, 128)
    base = s * share
    _zero_share_rows(zrows, out_sp, base, share)
    plsc.subcore_barrier()
    iota16 = lax.iota(jnp.int32, 16)

    def blk(j, _):
      e0 = wid * ept + j * 128
      pltpu.sync_copy(src_hbm.at[pl.ds(e0, 128)], sidx)
      pltpu.sync_copy(dst_hbm.at[pl.ds(e0, 128)], didx)
      ds_list = [pltpu.async_copy(hs_hbm.at[sidx], rows, gsem)]
      for h in range(HEADS):
        ds_list.append(pltpu.async_copy(den[h].at[didx], denb.at[h], gsem))
        pltpu.sync_copy(ex_hbm.at[pl.ds(h * epad + e0, 128)], exb.at[h])
      for d in ds_list:
        d.wait()
      for h in range(HEADS):
        for kk in range(8):
          sl = pl.ds(kk * 16, 16)
          attnT[pl.ds(h * 128 + kk * 16, 16)] = exb[h, sl] / denb[h, sl]
      def grp(g, _):
        avs = [attnT[pl.ds(h * 128 + g * 16, 16)] for h in range(HEADS)]
        def rw(l, _):
          i = g * 16 + l
          onehot = (iota16 == jnp.broadcast_to(l, (16,))).astype(jnp.float32)
          for h in range(HEADS):
            scv = jnp.broadcast_to(jnp.sum(avs[h] * onehot), (16,))
            for kk in range(2):
              c0 = h * 32 + kk * 16
              rows[i, pl.ds(c0, 16)] = rows[i, pl.ds(c0, 16)] * scv
          return 0
        lax.fori_loop(0, 16, rw, 0)
        return 0
      lax.fori_loop(0, 8, grp, 0)
      pltpu.sync_copy(rows, out_sp.at[didx], add=True)
      return 0
    lax.fori_loop(0, nblk, blk, 0)
    plsc.subcore_barrier()
    _sp_to_hbm(out_sp, lambda o, n: out_hbm.at[c, pl.ds(o, n)],
               rows, base, share, 128)

  return k(src_pad, dst_pad, ex, *den_h, hs)


def _sc_gcn(src_pad, dst_pad, xln):
  @functools.partial(
      pl.kernel,
      out_type=jax.ShapeDtypeStruct((2, NPASS, CHR, HID), jnp.float32),
      mesh=_mesh(),
      compiler_params=pltpu.CompilerParams(needs_layout_passes=False),
      scratch_types=[
          pltpu.VMEM((EPT2,), jnp.int32),
          pltpu.VMEM((EPT2,), jnp.int32),
          pltpu.VMEM((CPK_CAP + 16,), jnp.int32),
          pltpu.VMEM((128,), jnp.int32),
          pltpu.VMEM((128,), jnp.int32),
          pltpu.VMEM((128, 128), jnp.float32),
          pltpu.VMEM_SHARED((CHR, HID), jnp.float32),
          pltpu.SemaphoreType.DMA,
      ],
  )
  def k(src_hbm, dst_hbm, xln_hbm, g_out,
        sbuf, dbuf, cpk, sidx, didx, rows, chunk, gsem):
    c = lax.axis_index("c")
    s = lax.axis_index("s")
    wid = s * NCORES + c
    pltpu.sync_copy(src_hbm.at[pl.ds(wid * EPT2, EPT2)], sbuf)
    pltpu.sync_copy(dst_hbm.at[pl.ds(wid * EPT2, EPT2)], dbuf)
    base = s * SHARE_G
    iota = lax.iota(jnp.int32, 16)
    trash_pk = GTRASH * 131072

    def do_pass(p, _):
      lov = jnp.broadcast_to(p * CHUNK, (16,))
      hiv = lov + CHUNK
      _zero_rows_buf(rows, 128)
      _zero_share_rows(rows, chunk, base, SHARE_G)
      plsc.subcore_barrier()

      def cvec(v, cnt):
        off = v * 16
        srcv = sbuf[pl.ds(off, 16)]
        dstv = dbuf[pl.ds(off, 16)]
        m = (dstv >= lov) & (dstv < hiv)
        packed = (dstv - lov) * 131072 + srcv
        cum = plsc.cumsum(m.astype(jnp.int32))
        pos = jnp.broadcast_to(cnt, (16,)) + cum - 1
        pos = jnp.where(m, pos, CPK_CAP + iota)
        plsc.store_scatter(cpk, [pos], packed)
        return cnt + cum[15]
      cnt = lax.fori_loop(0, NV2, cvec, jnp.int32(0))

      cntv = jnp.broadcast_to(cnt, (16,))
      for t in range(8):
        plsc.store_scatter(cpk, [cntv + iota + t * 16],
                           jnp.full((16,), trash_pk, jnp.int32))
      nblk = (cnt + 127) // 128

      def gs(j, _):
        def up(kk, _):
          pv = cpk[pl.ds(j * 128 + kk * 16, 16)]
          sidx[pl.ds(kk * 16, 16)] = pv & 131071
          didx[pl.ds(kk * 16, 16)] = lax.shift_right_logical(pv, 17)
          return 0
        lax.fori_loop(0, 8, up, 0)
        pltpu.async_copy(xln_hbm.at[sidx], rows, gsem).wait()
        pltpu.sync_copy(rows, chunk.at[didx], add=True)
        return 0
      lax.fori_loop(0, nblk, gs, 0)
      plsc.subcore_barrier()
      _sp_to_hbm(chunk, lambda o, n: g_out.at[c, p, pl.ds(o, n)],
                 rows, base, SHARE_G, 128)
      plsc.subcore_barrier()
      return 0
    lax.fori_loop(0, NPASS, do_pass, 0)

  return k(src_pad, dst_pad, xln)


def _att_mat(att):
  r = jnp.arange(HID)
  return jnp.zeros((HID, 16), jnp.float32).at[r, r // CH].set(att.reshape(-1))


def _pad_edges(ei, epad, trash_dst):
  e = ei.shape[1]
  src = jnp.concatenate([ei[0], jnp.zeros((epad - e,), jnp.int32)])
  dst = jnp.concatenate([ei[1], jnp.full((epad - e,), trash_dst, jnp.int32)])
  return src, dst


def kernel(x_building, x_cable_group, x_transformer, edge_index_b2c,
           edge_index_c2t, edge_index_b2b, W_src_bl, W_dst_bl, att_src_bl,
           att_dst_bl, bias_bl, W_src_lt, W_dst_lt, att_src_lt, att_dst_lt,
           bias_lt, W_gcn, b_gcn):
  src1, dst1 = _pad_edges(edge_index_b2c, EPAD1, NC)
  src3, dst3 = _pad_edges(edge_index_c2t, EPAD3, NT)
  src2, dst2 = _pad_edges(edge_index_b2b, EPAD2, NB)

  degp = _sc_deg(dst2).reshape(2, DEG_N)
  deg = degp[0, :NB] + degp[1, :NB] + 1.0
  dis = lax.rsqrt(deg).reshape(NB, 1)

  a_s_bl = _att_mat(att_src_bl)
  a_d_bl = _att_mat(att_dst_bl)
  hs_b, asrc16_b, xln = _tc_prep_b(x_building, dis, W_src_bl, a_s_bl, W_gcn)

  adst16_c = _tc_attdst(x_cable_group, W_dst_bl, a_d_bl)
  adst16_c = jnp.concatenate(
      [adst16_c, jnp.zeros((ND8_C - NC, 16), jnp.float32)])
  asrc_h_b = tuple(asrc16_b[:, h] for h in range(HEADS))
  adst_h_c = tuple(adst16_c[:, h] for h in range(HEADS))

  ex1, denp1 = _sc_gat_den(src1, dst1, asrc_h_b, adst_h_c,
                           EPAD1, EPT1, NBLK1, ND8_C)
  den1 = _tc_den_combine(denp1.reshape(2, HEADS, ND8_C))
  den1_h = tuple(den1[h] for h in range(HEADS))
  outc_p = _sc_gat_agg(src1, dst1, ex1, den1_h, hs_b,
                       EPAD1, EPT1, NBLK1, ND8_C)
  h_c, hs_c, asrc16_c = _tc_combine_c(
      x_cable_group, outc_p[:, :NC], bias_bl.reshape(1, HID), W_src_lt,
      _att_mat(att_src_lt))

  adst16_t = _tc_attdst(x_transformer, W_dst_lt, _att_mat(att_dst_lt))
  adst16_t = jnp.concatenate(
      [adst16_t, jnp.zeros((ND8_T - NT, 16), jnp.float32)])
  asrc_h_c = tuple(asrc16_c[:, h] for h in range(HEADS))
  adst_h_t = tuple(adst16_t[:, h] for h in range(HEADS))
  ex3, denp3 = _sc_gat_den(src3, dst3, asrc_h_c, adst_h_t,
                           EPAD3, EPT3, NBLK3, ND8_T)
  den3 = _tc_den_combine(denp3.reshape(2, HEADS, ND8_T))
  den3_h = tuple(den3[h] for h in range(HEADS))
  outt_p = _sc_gat_agg(src3, dst3, ex3, den3_h, hs_c,
                       EPAD3, EPT3, NBLK3, ND8_T)
  h_t = _tc_final_t(x_transformer, outt_p, bias_lt.reshape(1, HID))

  g = _sc_gcn(src2, dst2, xln)
  h_b = _tc_final_b(x_building, xln, dis, g, b_gcn.reshape(1, HID))

  return (h_b, h_c, h_t)


# speedup vs baseline: 17.5063x; 1.5674x over previous
import functools

import jax
import jax.numpy as jnp
from jax import lax
from jax.experimental import pallas as pl
from jax.experimental.pallas import tpu as pltpu
from jax.experimental.pallas import tpu_sc as plsc

HID = 128
HEADS = 4
CH = 32
NB = 100000
NC = 10000
NT = 1000

NCORES = 2
NSUB = 16
NTILES = NCORES * NSUB

E1 = 100000
EPT1 = 3200
NBLK1 = EPT1 // 128
EPAD1 = EPT1 * NTILES
ND8_C = 10112
SHARE_C = ND8_C // NSUB

E3 = 10000
EPT3 = 384
NBLK3 = EPT3 // 128
EPAD3 = EPT3 * NTILES
ND8_T = 1024
SHARE_T = ND8_T // NSUB

E2 = 600000
EPT2 = 18816
NV2 = EPT2 // 16
EPAD2 = EPT2 * NTILES
NPASS = 14
CHUNK = 7152
CHR = 7168
GTRASH = 7152
SHARE_G = CHR // NSUB
CPK_CAP = EPT2 + 128

DEG_N = 100096
SHARE_D = DEG_N // NSUB


def _prep_b_body(xb_ref, dis_ref, ws_ref, as_ref, wg_ref, hs_ref, a16_ref,
                 xln_ref):
  xb = xb_ref[...]
  hs = jnp.dot(xb, ws_ref[...], preferred_element_type=jnp.float32)
  hs_ref[...] = hs
  a16_ref[...] = jnp.dot(hs, as_ref[...], preferred_element_type=jnp.float32)
  xl = jnp.dot(xb, wg_ref[...], preferred_element_type=jnp.float32)
  xln_ref[...] = xl * dis_ref[...]


def _tc_prep_b(xb, dis, wsrc, a_s, wgcn):
  br = 1000
  return pl.pallas_call(
      _prep_b_body,
      grid=(NB // br,),
      in_specs=[
          pl.BlockSpec((br, HID), lambda i: (i, 0)),
          pl.BlockSpec((br, 1), lambda i: (i, 0)),
          pl.BlockSpec((HID, HID), lambda i: (0, 0)),
          pl.BlockSpec((HID, 16), lambda i: (0, 0)),
          pl.BlockSpec((HID, HID), lambda i: (0, 0)),
      ],
      out_specs=[
          pl.BlockSpec((br, HID), lambda i: (i, 0)),
          pl.BlockSpec((br, 16), lambda i: (i, 0)),
          pl.BlockSpec((br, HID), lambda i: (i, 0)),
      ],
      out_shape=[
          jax.ShapeDtypeStruct((NB, HID), jnp.float32),
          jax.ShapeDtypeStruct((NB, 16), jnp.float32),
          jax.ShapeDtypeStruct((NB, HID), jnp.float32),
      ],
  )(xb, dis, wsrc, a_s, wgcn)


def _attdst_body(x_ref, w_ref, a_ref, o_ref):
  h = jnp.dot(x_ref[...], w_ref[...], preferred_element_type=jnp.float32)
  o_ref[...] = jnp.dot(h, a_ref[...], preferred_element_type=jnp.float32)


def _tc_attdst(x, w, a16):
  n = x.shape[0]
  br = 1000
  return pl.pallas_call(
      _attdst_body,
      grid=(n // br,),
      in_specs=[
          pl.BlockSpec((br, HID), lambda i: (i, 0)),
          pl.BlockSpec((HID, HID), lambda i: (0, 0)),
          pl.BlockSpec((HID, 16), lambda i: (0, 0)),
      ],
      out_specs=pl.BlockSpec((br, 16), lambda i: (i, 0)),
      out_shape=jax.ShapeDtypeStruct((n, 16), jnp.float32),
  )(x, w, a16)


def _denc_body(p_ref, o_ref):
  o_ref[...] = p_ref[0] + p_ref[1] + 1e-16


def _tc_den_combine(p):
  nd8 = p.shape[2]
  return pl.pallas_call(
      _denc_body,
      grid=(1,),
      in_specs=[pl.BlockSpec((2, HEADS, nd8), lambda i: (0, 0, 0))],
      out_specs=pl.BlockSpec((HEADS, nd8), lambda i: (0, 0)),
      out_shape=jax.ShapeDtypeStruct((HEADS, nd8), jnp.float32),
  )(p)


def _comb_c_body(xc_ref, p_ref, b_ref, w_ref, a_ref, hc_ref, hs_ref, a16_ref):
  hc = xc_ref[...] + 0.5 * (p_ref[0] + p_ref[1] + b_ref[...])
  hc_ref[...] = hc
  hs = jnp.dot(hc, w_ref[...], preferred_element_type=jnp.float32)
  hs_ref[...] = hs
  a16_ref[...] = jnp.dot(hs, a_ref[...], preferred_element_type=jnp.float32)


def _tc_combine_c(xc, p, bias, wsrc, a_s):
  br = 1000
  return pl.pallas_call(
      _comb_c_body,
      grid=(NC // br,),
      in_specs=[
          pl.BlockSpec((br, HID), lambda i: (i, 0)),
          pl.BlockSpec((2, br, HID), lambda i: (0, i, 0)),
          pl.BlockSpec((1, HID), lambda i: (0, 0)),
          pl.BlockSpec((HID, HID), lambda i: (0, 0)),
          pl.BlockSpec((HID, 16), lambda i: (0, 0)),
      ],
      out_specs=[
          pl.BlockSpec((br, HID), lambda i: (i, 0)),
          pl.BlockSpec((br, HID), lambda i: (i, 0)),
          pl.BlockSpec((br, 16), lambda i: (i, 0)),
      ],
      out_shape=[
          jax.ShapeDtypeStruct((NC, HID), jnp.float32),
          jax.ShapeDtypeStruct((NC, HID), jnp.float32),
          jax.ShapeDtypeStruct((NC, 16), jnp.float32),
      ],
  )(xc, p, bias, wsrc, a_s)


def _final_t_body(xt_ref, q_ref, b_ref, o_ref):
  o_ref[...] = xt_ref[...] + 0.5 * (q_ref[0] + q_ref[1] + b_ref[...])


def _tc_final_t(xt, q, bias):
  return pl.pallas_call(
      _final_t_body,
      grid=(1,),
      in_specs=[
          pl.BlockSpec((NT, HID), lambda i: (0, 0)),
          pl.BlockSpec((2, NT, HID), lambda i: (0, 0, 0)),
          pl.BlockSpec((1, HID), lambda i: (0, 0)),
      ],
      out_specs=pl.BlockSpec((NT, HID), lambda i: (0, 0)),
      out_shape=jax.ShapeDtypeStruct((NT, HID), jnp.float32),
  )(xt, q, bias)


def _final_b_body(xb_ref, xln_ref, dis_ref, g_ref, bg_ref, o_ref):
  g = g_ref[0, 0] + g_ref[1, 0]
  o_ref[...] = xb_ref[...] + 0.2 * (
      dis_ref[...] * (g + xln_ref[...]) + bg_ref[...])


def _tc_final_b(xb, xln, dis, g, bg):
  br = 3576
  nj = CHUNK // br
  return pl.pallas_call(
      _final_b_body,
      grid=(NPASS, nj),
      in_specs=[
          pl.BlockSpec((br, HID), lambda i, j: (i * nj + j, 0)),
          pl.BlockSpec((br, HID), lambda i, j: (i * nj + j, 0)),
          pl.BlockSpec((br, 1), lambda i, j: (i * nj + j, 0)),
          pl.BlockSpec((2, 1, br, HID), lambda i, j: (0, i, j, 0)),
          pl.BlockSpec((1, HID), lambda i, j: (0, 0)),
      ],
      out_specs=pl.BlockSpec((br, HID), lambda i, j: (i * nj + j, 0)),
      out_shape=jax.ShapeDtypeStruct((NB, HID), jnp.float32),
  )(xb, xln, dis, g, bg)


def _mesh():
  return plsc.VectorSubcoreMesh(core_axis_name="c", subcore_axis_name="s")


def _zero_vec_buf(ref, n):
  z = jnp.zeros((16,), jnp.float32)
  def body(i, _):
    ref[pl.ds(i * 16, 16)] = z
    return 0
  lax.fori_loop(0, n // 16, body, 0)


def _zero_rows_buf(ref, rows, width=128):
  z = jnp.zeros((16,), jnp.float32)
  w = width // 16
  def body(i, _):
    r = i // w
    k = i % w
    ref[r, pl.ds(k * 16, 16)] = z
    return 0
  lax.fori_loop(0, rows * w, body, 0)


def _zero_share_rows(zsrc, dst, base, share):
  full, rem = divmod(share, 128)
  for t in range(full):
    pltpu.sync_copy(zsrc, dst.at[pl.ds(base + t * 128, 128)])
  if rem:
    pltpu.sync_copy(zsrc.at[pl.ds(0, rem)], dst.at[pl.ds(base + full * 128, rem)])


def _zero_share_1d(zbuf, sp, base, share):
  full, rem = divmod(share, 2048)
  for t in range(full):
    pltpu.sync_copy(zbuf, sp.at[pl.ds(base + t * 2048, 2048)])
  if rem:
    pltpu.sync_copy(zbuf.at[pl.ds(0, rem)],
                    sp.at[pl.ds(base + full * 2048, rem)])


def _sp_to_hbm(sp_ref, out_slice, bounce, base, share, brows):
  full, rem = divmod(share, brows)
  for t in range(full):
    o = base + t * brows
    pltpu.sync_copy(sp_ref.at[pl.ds(o, brows)], bounce)
    pltpu.sync_copy(bounce, out_slice(o, brows))
  if rem:
    o = base + full * brows
    pltpu.sync_copy(sp_ref.at[pl.ds(o, rem)], bounce.at[pl.ds(0, rem)])
    pltpu.sync_copy(bounce.at[pl.ds(0, rem)], out_slice(o, rem))


def _sc_deg(dst_pad):
  @functools.partial(
      pl.kernel,
      out_type=jax.ShapeDtypeStruct((2 * DEG_N,), jnp.float32),
      mesh=_mesh(),
      compiler_params=pltpu.CompilerParams(needs_layout_passes=False),
      scratch_types=[
          pltpu.VMEM((EPT2,), jnp.int32),
          pltpu.VMEM((128,), jnp.float32),
          pltpu.VMEM((128,), jnp.int32),
          pltpu.VMEM((2048,), jnp.float32),
          pltpu.VMEM_SHARED((DEG_N,), jnp.float32),
      ],
  )
  def k(dst_hbm, deg_out, dbuf, ones_v, didx, zbuf, deg_sp):
    c = lax.axis_index("c")
    s = lax.axis_index("s")
    wid = s * NCORES + c
    pltpu.sync_copy(dst_hbm.at[pl.ds(wid * EPT2, EPT2)], dbuf)
    _zero_vec_buf(zbuf, 2048)
    one = jnp.ones((16,), jnp.float32)
    def ob(i, _):
      ones_v[pl.ds(i * 16, 16)] = one
      return 0
    lax.fori_loop(0, 8, ob, 0)
    base = s * SHARE_D
    for t in range(SHARE_D // 2048):
      pltpu.sync_copy(zbuf, deg_sp.at[pl.ds(base + t * 2048, 2048)])
    rem = SHARE_D % 2048
    if rem:
      pltpu.sync_copy(zbuf.at[pl.ds(0, rem)],
                      deg_sp.at[pl.ds(base + (SHARE_D // 2048) * 2048, rem)])
    plsc.subcore_barrier()
    def blk(j, _):
      def cp(kk, _):
        didx[pl.ds(kk * 16, 16)] = dbuf[pl.ds(j * 128 + kk * 16, 16)]
        return 0
      lax.fori_loop(0, 8, cp, 0)
      pltpu.sync_copy(ones_v, deg_sp.at[didx], add=True)
      return 0
    lax.fori_loop(0, EPT2 // 128, blk, 0)
    plsc.subcore_barrier()
    _sp_to_hbm(deg_sp, lambda o, n: deg_out.at[pl.ds(c * DEG_N + o, n)],
               zbuf, base, SHARE_D, 2048)

  return k(dst_pad)


def _sc_gat_den(src_pad, dst_pad, asrc_h, adst_h, epad, ept, nblk, nd8):
  share = nd8 // NSUB

  @functools.partial(
      pl.kernel,
      out_type=(
          jax.ShapeDtypeStruct((HEADS * epad,), jnp.float32),
          jax.ShapeDtypeStruct((2 * HEADS * nd8,), jnp.float32),
      ),
      mesh=_mesh(),
      compiler_params=pltpu.CompilerParams(needs_layout_passes=False),
      scratch_types=[
          pltpu.VMEM((128,), jnp.int32),
          pltpu.VMEM((128,), jnp.int32),
          pltpu.VMEM((HEADS, 128), jnp.float32),
          pltpu.VMEM((HEADS, 128), jnp.float32),
          pltpu.VMEM((HEADS, 128), jnp.float32),
          pltpu.VMEM((128,), jnp.int32),
          pltpu.VMEM((2048,), jnp.float32),
          pltpu.VMEM_SHARED((HEADS * nd8,), jnp.float32),
          pltpu.SemaphoreType.DMA,
      ],
  )
  def k(src_hbm, dst_hbm, as0, as1, as2, as3, ad0, ad1, ad2, ad3,
        ex_out, den_out, sidx, didx, asg, adg, exb, didxo, zbuf, den_sp,
        gsem):
    asrc = (as0, as1, as2, as3)
    adst = (ad0, ad1, ad2, ad3)
    c = lax.axis_index("c")
    s = lax.axis_index("s")
    wid = s * NCORES + c
    _zero_vec_buf(zbuf, 2048)
    share2 = HEADS * share
    base2 = s * share2
    _zero_share_1d(zbuf, den_sp, base2, share2)
    plsc.subcore_barrier()

    def blk(j, _):
      e0 = wid * ept + j * 128
      pltpu.sync_copy(src_hbm.at[pl.ds(e0, 128)], sidx)
      pltpu.sync_copy(dst_hbm.at[pl.ds(e0, 128)], didx)
      ds_list = []
      for h in range(HEADS):
        ds_list.append(pltpu.async_copy(asrc[h].at[sidx], asg.at[h], gsem))
        ds_list.append(pltpu.async_copy(adst[h].at[didx], adg.at[h], gsem))
      for d in ds_list:
        d.wait()
      for h in range(HEADS):
        for kk in range(8):
          sl = pl.ds(kk * 16, 16)
          al = asg[h, sl] + adg[h, sl]
          al = jnp.where(al >= 0, al, 0.2 * al)
          exb[h, sl] = jnp.exp(al)
      for h in range(HEADS):
        pltpu.sync_copy(exb.at[h], ex_out.at[pl.ds(h * epad + e0, 128)])
        def off(kk, _, h=h):
          sl = pl.ds(kk * 16, 16)
          didxo[sl] = didx[sl] + h * nd8
          return 0
        lax.fori_loop(0, 8, off, 0)
        pltpu.sync_copy(exb.at[h], den_sp.at[didxo], add=True)
      return 0
    lax.fori_loop(0, nblk, blk, 0)
    plsc.subcore_barrier()
    _sp_to_hbm(den_sp,
               lambda o, n: den_out.at[pl.ds(c * HEADS * nd8 + o, n)],
               zbuf, base2, share2, 2048)

  return k(src_pad, dst_pad, *asrc_h, *adst_h)


def _sc_gat_agg(src_pad, dst_pad, ex, den_h, hs, epad, ept, nblk, nd8):
  share = nd8 // NSUB

  @functools.partial(
      pl.kernel,
      out_type=jax.ShapeDtypeStruct((2, nd8, HID), jnp.float32),
      mesh=_mesh(),
      compiler_params=pltpu.CompilerParams(needs_layout_passes=False),
      scratch_types=[
          pltpu.VMEM((128,), jnp.int32),
          pltpu.VMEM((128,), jnp.int32),
          pltpu.VMEM((HEADS, 128), jnp.float32),
          pltpu.VMEM((HEADS, 128), jnp.float32),
          pltpu.VMEM((HEADS * 128,), jnp.float32),
          pltpu.VMEM((128, 128), jnp.float32),
          pltpu.VMEM((128, 128), jnp.float32),
          pltpu.VMEM_SHARED((nd8, HID), jnp.float32),
          pltpu.SemaphoreType.DMA,
      ],
  )
  def k(src_hbm, dst_hbm, ex_hbm, dn0, dn1, dn2, dn3, hs_hbm, out_hbm,
        sidx, didx, exb, denb, attnT, rows, zrows, out_sp, gsem):
    den = (dn0, dn1, dn2, dn3)
    c = lax.axis_index("c")
    s = lax.axis_index("s")
    wid = s * NCORES + c
    _zero_rows_buf(zrows, 128)
    base = s * share
    _zero_share_rows(zrows, out_sp, base, share)
    plsc.subcore_barrier()
    iota16 = lax.iota(jnp.int32, 16)

    def blk(j, _):
      e0 = wid * ept + j * 128
      pltpu.sync_copy(src_hbm.at[pl.ds(e0, 128)], sidx)
      pltpu.sync_copy(dst_hbm.at[pl.ds(e0, 128)], didx)
      ds_list = [pltpu.async_copy(hs_hbm.at[sidx], rows, gsem)]
      for h in range(HEADS):
        ds_list.append(pltpu.async_copy(den[h].at[didx], denb.at[h], gsem))
        pltpu.sync_copy(ex_hbm.at[pl.ds(h * epad + e0, 128)], exb.at[h])
      for d in ds_list:
        d.wait()
      for h in range(HEADS):
        for kk in range(8):
          sl = pl.ds(kk * 16, 16)
          attnT[pl.ds(h * 128 + kk * 16, 16)] = exb[h, sl] / denb[h, sl]
      def grp(g, _):
        avs = [attnT[pl.ds(h * 128 + g * 16, 16)] for h in range(HEADS)]
        def rw(l, _):
          i = g * 16 + l
          onehot = (iota16 == jnp.broadcast_to(l, (16,))).astype(jnp.float32)
          for h in range(HEADS):
            scv = jnp.broadcast_to(jnp.sum(avs[h] * onehot), (16,))
            for kk in range(2):
              c0 = h * 32 + kk * 16
              rows[i, pl.ds(c0, 16)] = rows[i, pl.ds(c0, 16)] * scv
          return 0
        lax.fori_loop(0, 16, rw, 0)
        return 0
      lax.fori_loop(0, 8, grp, 0)
      pltpu.sync_copy(rows, out_sp.at[didx], add=True)
      return 0
    lax.fori_loop(0, nblk, blk, 0)
    plsc.subcore_barrier()
    _sp_to_hbm(out_sp, lambda o, n: out_hbm.at[c, pl.ds(o, n)],
               rows, base, share, 128)

  return k(src_pad, dst_pad, ex, *den_h, hs)


def _sc_gcn(src_pad, dst_pad, xln):
  @functools.partial(
      pl.kernel,
      out_type=jax.ShapeDtypeStruct((2, NPASS, CHR, HID), jnp.float32),
      mesh=_mesh(),
      compiler_params=pltpu.CompilerParams(needs_layout_passes=False),
      scratch_types=[
          pltpu.VMEM((EPT2,), jnp.int32),
          pltpu.VMEM((CPK_CAP + 16,), jnp.int32),
          pltpu.VMEM((128,), jnp.int32),
          pltpu.VMEM((128,), jnp.int32),
          pltpu.VMEM((128,), jnp.int32),
          pltpu.VMEM((128,), jnp.int32),
          pltpu.VMEM((128,), jnp.int32),
          pltpu.VMEM((128,), jnp.int32),
          pltpu.VMEM((128, 128), jnp.float32),
          pltpu.VMEM((128, 128), jnp.float32),
          pltpu.VMEM_SHARED((CHR, HID), jnp.float32),
          pltpu.SemaphoreType.DMA,
          pltpu.SemaphoreType.DMA,
          pltpu.SemaphoreType.DMA,
          pltpu.SemaphoreType.DMA,
          pltpu.SemaphoreType.DMA,
          pltpu.SemaphoreType.DMA,
      ],
  )
  def k(src_hbm, dst_hbm, xln_hbm, g_out,
        dbuf, cpk, gi_a, gi_b, sidx_a, sidx_b, didx_a, didx_b,
        rows_a, rows_b, chunk,
        sem_g0, sem_g1, sem_s0, sem_s1, sem_c0, sem_c1):
    gi = (gi_a, gi_b)
    sidx = (sidx_a, sidx_b)
    didx = (didx_a, didx_b)
    rows = (rows_a, rows_b)
    sem_g = (sem_g0, sem_g1)
    sem_s = (sem_s0, sem_s1)
    sem_c = (sem_c0, sem_c1)
    c = lax.axis_index("c")
    s = lax.axis_index("s")
    wid = s * NCORES + c
    pltpu.sync_copy(dst_hbm.at[pl.ds(wid * EPT2, EPT2)], dbuf)
    base = s * SHARE_G
    iota = lax.iota(jnp.int32, 16)
    trash_pk = GTRASH * 32768
    widbase = jnp.broadcast_to(wid * EPT2, (16,))

    def unpack(j, b):
      def up(kk, _):
        pv = cpk[pl.ds(j * 128 + kk * 16, 16)]
        gi[b][pl.ds(kk * 16, 16)] = (pv & 32767) + widbase
        didx[b][pl.ds(kk * 16, 16)] = lax.shift_right_logical(pv, 15)
        return 0
      lax.fori_loop(0, 8, up, 0)

    def do_pass(p, _):
      lov = jnp.broadcast_to(p * CHUNK, (16,))
      hiv = lov + CHUNK
      _zero_rows_buf(rows_a, 128)
      _zero_share_rows(rows_a, chunk, base, SHARE_G)
      plsc.subcore_barrier()

      def cvec(v, cnt):
        off = v * 16
        dstv = dbuf[pl.ds(off, 16)]
        m = (dstv >= lov) & (dstv < hiv)
        eidxv = jnp.broadcast_to(off, (16,)) + iota
        packed = (dstv - lov) * 32768 + eidxv
        cum = plsc.cumsum(m.astype(jnp.int32))
        pos = jnp.broadcast_to(cnt, (16,)) + cum - 1
        pos = jnp.where(m, pos, CPK_CAP + iota)
        plsc.store_scatter(cpk, [pos], packed)
        return cnt + cum[15]
      cnt = lax.fori_loop(0, NV2, cvec, jnp.int32(0))

      cntv = jnp.broadcast_to(cnt, (16,))
      for t in range(8):
        plsc.store_scatter(cpk, [cntv + iota + t * 16],
                           jnp.full((16,), trash_pk, jnp.int32))
      nblk = (cnt + 127) // 128

      @pl.when(nblk > 0)
      def _():
        unpack(0, 0)
        pltpu.async_copy(src_hbm.at[gi[0]], sidx[0], sem_s[0]).wait()
        pltpu.async_copy(xln_hbm.at[sidx[0]], rows[0], sem_g[0])

      def gs(j, _):
        def stage(b):
          nb = 1 - b
          @pl.when(j >= 1)
          def _():
            pltpu.make_async_copy(rows[nb], chunk.at[didx[nb]],
                                  sem_c[nb]).wait()
          @pl.when(j + 1 < nblk)
          def _():
            unpack(j + 1, nb)
            pltpu.async_copy(src_hbm.at[gi[nb]], sidx[nb], sem_s[nb])
          pltpu.make_async_copy(xln_hbm.at[sidx[b]], rows[b],
                                sem_g[b]).wait()
          pltpu.async_copy(rows[b], chunk.at[didx[b]], sem_c[b], add=True)
          @pl.when(j + 1 < nblk)
          def _():
            pltpu.make_async_copy(src_hbm.at[gi[nb]], sidx[nb],
                                  sem_s[nb]).wait()
            pltpu.async_copy(xln_hbm.at[sidx[nb]], rows[nb], sem_g[nb])
        @pl.when(j % 2 == 0)
        def _():
          stage(0)
        @pl.when(j % 2 == 1)
        def _():
          stage(1)
        return 0
      lax.fori_loop(0, nblk, gs, 0)
      @pl.when(nblk > 0)
      def _():
        @pl.when((nblk - 1) % 2 == 0)
        def _():
          pltpu.make_async_copy(rows[0], chunk.at[didx[0]], sem_c[0]).wait()
        @pl.when((nblk - 1) % 2 == 1)
        def _():
          pltpu.make_async_copy(rows[1], chunk.at[didx[1]], sem_c[1]).wait()
      plsc.subcore_barrier()
      _sp_to_hbm(chunk, lambda o, n: g_out.at[c, p, pl.ds(o, n)],
                 rows_a, base, SHARE_G, 128)
      plsc.subcore_barrier()
      return 0
    lax.fori_loop(0, NPASS, do_pass, 0)

  return k(src_pad, dst_pad, xln)


def _att_mat(att):
  r = jnp.arange(HID)
  return jnp.zeros((HID, 16), jnp.float32).at[r, r // CH].set(att.reshape(-1))


def _pad_edges(ei, epad, trash_dst):
  e = ei.shape[1]
  src = jnp.concatenate([ei[0], jnp.zeros((epad - e,), jnp.int32)])
  dst = jnp.concatenate([ei[1], jnp.full((epad - e,), trash_dst, jnp.int32)])
  return src, dst


def kernel(x_building, x_cable_group, x_transformer, edge_index_b2c,
           edge_index_c2t, edge_index_b2b, W_src_bl, W_dst_bl, att_src_bl,
           att_dst_bl, bias_bl, W_src_lt, W_dst_lt, att_src_lt, att_dst_lt,
           bias_lt, W_gcn, b_gcn):
  src1, dst1 = _pad_edges(edge_index_b2c, EPAD1, NC)
  src3, dst3 = _pad_edges(edge_index_c2t, EPAD3, NT)
  src2, dst2 = _pad_edges(edge_index_b2b, EPAD2, NB)

  degp = _sc_deg(dst2).reshape(2, DEG_N)
  deg = degp[0, :NB] + degp[1, :NB] + 1.0
  dis = lax.rsqrt(deg).reshape(NB, 1)

  a_s_bl = _att_mat(att_src_bl)
  a_d_bl = _att_mat(att_dst_bl)
  hs_b, asrc16_b, xln = _tc_prep_b(x_building, dis, W_src_bl, a_s_bl, W_gcn)

  adst16_c = _tc_attdst(x_cable_group, W_dst_bl, a_d_bl)
  adst16_c = jnp.concatenate(
      [adst16_c, jnp.zeros((ND8_C - NC, 16), jnp.float32)])
  asrc_h_b = tuple(asrc16_b[:, h] for h in range(HEADS))
  adst_h_c = tuple(adst16_c[:, h] for h in range(HEADS))

  ex1, denp1 = _sc_gat_den(src1, dst1, asrc_h_b, adst_h_c,
                           EPAD1, EPT1, NBLK1, ND8_C)
  den1 = _tc_den_combine(denp1.reshape(2, HEADS, ND8_C))
  den1_h = tuple(den1[h] for h in range(HEADS))
  outc_p = _sc_gat_agg(src1, dst1, ex1, den1_h, hs_b,
                       EPAD1, EPT1, NBLK1, ND8_C)
  h_c, hs_c, asrc16_c = _tc_combine_c(
      x_cable_group, outc_p[:, :NC], bias_bl.reshape(1, HID), W_src_lt,
      _att_mat(att_src_lt))

  adst16_t = _tc_attdst(x_transformer, W_dst_lt, _att_mat(att_dst_lt))
  adst16_t = jnp.concatenate(
      [adst16_t, jnp.zeros((ND8_T - NT, 16), jnp.float32)])
  asrc_h_c = tuple(asrc16_c[:, h] for h in range(HEADS))
  adst_h_t = tuple(adst16_t[:, h] for h in range(HEADS))
  ex3, denp3 = _sc_gat_den(src3, dst3, asrc_h_c, adst_h_t,
                           EPAD3, EPT3, NBLK3, ND8_T)
  den3 = _tc_den_combine(denp3.reshape(2, HEADS, ND8_T))
  den3_h = tuple(den3[h] for h in range(HEADS))
  outt_p = _sc_gat_agg(src3, dst3, ex3, den3_h, hs_c,
                       EPAD3, EPT3, NBLK3, ND8_T)
  h_t = _tc_final_t(x_transformer, outt_p, bias_lt.reshape(1, HID))

  g = _sc_gcn(src2, dst2, xln)
  h_b = _tc_final_b(x_building, xln, dis, g, b_gcn.reshape(1, HID))

  return (h_b, h_c, h_t)
